# baseline (speedup 1.0000x reference)
import functools

import jax
import jax.numpy as jnp
from jax import lax
from jax.experimental import pallas as pl
from jax.experimental.pallas import tpu as pltpu
from jax.experimental.pallas import tpu_sc as plsc

_B, _P, _K, _OUT = 16, 1024, 20, 40
_N = _B * _P
_NEG = -3e38
_NWORKERS = 32


def _edge_topk_kernel(go, x_ref, wa_ref, wb_ref, b_ref, idx_ref, y_ref,
                      c_ref):
    bidx = pl.program_id(0)
    x = x_ref[0]
    n2 = jnp.sum(x * x, axis=1, keepdims=True)
    gram = lax.dot_general(x, x, (((1,), (1,)), ((), ())),
                           preferred_element_type=jnp.float32)
    ones_row = jnp.ones((1, x.shape[1]), jnp.float32)
    n2r = lax.dot_general(ones_row, x * x, (((1,), (1,)), ((), ())),
                          preferred_element_type=jnp.float32)
    dist = n2 + n2r - 2.0 * gram

    wb = wb_ref[...]
    o = wb.shape[1]
    y = lax.dot_general(x, wb, (((1,), (0,)), ((), ())),
                        preferred_element_type=jnp.float32)
    if go != o:
        y = jnp.concatenate([y, jnp.zeros((_P, go - o), jnp.float32)], axis=1)
    y_ref[0] = y
    c_ref[0] = lax.dot_general(x, wa_ref[...] - wb, (((1,), (0,)), ((), ())),
                               preferred_element_type=jnp.float32) + b_ref[...]

    col = lax.broadcasted_iota(jnp.int32, (_P, _P), 1)
    lane = lax.broadcasted_iota(jnp.int32, (_P, _K), 1)
    v = jnp.minimum(dist, 254.0) + 1.0
    ib = lax.bitcast_convert_type(v, jnp.int32)
    key = (((ib - 0x3F800000) << 4) & ~1023) | col
    idx_acc = jnp.zeros((_P, _K), jnp.int32)
    for k in range(_K):
        kmin = jnp.min(key, axis=1, keepdims=True)
        aidx = kmin & 1023
        idx_acc = jnp.where(lane == k, aidx, idx_acc)
        key = jnp.where(key == kmin, jnp.int32(0x7FFFFFFF), key)
    idx_ref[0] = idx_acc + bidx * _P


def _edge_topk(x, wa, wb, b, h_off, bs):
    d, o = wa.shape
    go = max(o, 128)
    return pl.pallas_call(
        functools.partial(_edge_topk_kernel, go),
        grid=(bs,),
        in_specs=[
            pl.BlockSpec((1, _P, d), lambda i: (h_off + i, 0, 0)),
            pl.BlockSpec((d, o), lambda i: (0, 0)),
            pl.BlockSpec((d, o), lambda i: (0, 0)),
            pl.BlockSpec((1, o), lambda i: (0, 0)),
        ],
        out_specs=[
            pl.BlockSpec((1, _P, _K), lambda i: (i, 0, 0)),
            pl.BlockSpec((1, _P, go), lambda i: (i, 0, 0)),
            pl.BlockSpec((1, _P, o), lambda i: (i, 0, 0)),
        ],
        out_shape=[
            jax.ShapeDtypeStruct((bs, _P, _K), jnp.int32),
            jax.ShapeDtypeStruct((bs, _P, go), jnp.float32),
            jax.ShapeDtypeStruct((bs, _P, o), jnp.float32),
        ],
    )(x, wa, wb, b)


def _sc_gather(y_flat, idx_flat):
    n_idx = idx_flat.shape[0]
    o = y_flat.shape[1]
    per_w = n_idx // _NWORKERS
    rc = max(8, min(512, (160 * 1024) // (o * 4)))
    while per_w % rc or (per_w // rc) % 2:
        rc //= 2
    n_chunks = per_w // rc
    mesh = plsc.VectorSubcoreMesh(core_axis_name="c", subcore_axis_name="s")

    @functools.partial(
        pl.kernel,
        mesh=mesh,
        out_type=jax.ShapeDtypeStruct((n_idx, o), jnp.float32),
        scratch_types=[
            pltpu.VMEM((per_w,), jnp.int32),
            pltpu.VMEM((rc, o), jnp.float32),
            pltpu.VMEM((rc, o), jnp.float32),
            pltpu.SemaphoreType.DMA,
            pltpu.SemaphoreType.DMA,
        ],
    )
    def gather_kernel(y_hbm, idx_hbm, out_hbm, idx_v, buf_a, buf_b,
                      sem_a, sem_b):
        wid = lax.axis_index("s") * 2 + lax.axis_index("c")
        base = wid * per_w
        pltpu.sync_copy(idx_hbm.at[pl.ds(base, per_w)], idx_v)

        def start(c, buf, sem):
            pltpu.async_copy(y_hbm.at[idx_v.at[pl.ds(c * rc, rc)]], buf, sem)

        def finish(c, buf, sem):
            pltpu.make_async_copy(
                y_hbm.at[idx_v.at[pl.ds(c * rc, rc)]], buf, sem).wait()
            pltpu.sync_copy(buf, out_hbm.at[pl.ds(base + c * rc, rc)])

        start(0, buf_a, sem_a)

        @pl.loop(0, n_chunks // 2)
        def _(t):
            c0 = 2 * t
            start(c0 + 1, buf_b, sem_b)
            finish(c0, buf_a, sem_a)

            @pl.when(c0 + 2 < n_chunks)
            def _():
                start(c0 + 2, buf_a, sem_a)

            finish(c0 + 1, buf_b, sem_b)

    return gather_kernel(y_flat, idx_flat)


def _nbr_max_kernel(o, go, yg_ref, c_ref, out_ref):
    yg = yg_ref[...]
    m = yg[:, 0:o]
    for k in range(1, _K):
        m = jnp.maximum(m, yg[:, k * go:k * go + o])
    out_ref[...] = m + c_ref[...]


def _nbr_max(ygat, c_flat, go):
    n, o = c_flat.shape
    rows = 512
    return pl.pallas_call(
        functools.partial(_nbr_max_kernel, o, go),
        grid=(n // rows,),
        in_specs=[
            pl.BlockSpec((rows, _K * go), lambda i: (i, 0)),
            pl.BlockSpec((rows, o), lambda i: (i, 0)),
        ],
        out_specs=pl.BlockSpec((rows, o), lambda i: (i, 0)),
        out_shape=jax.ShapeDtypeStruct((n, o), jnp.float32),
    )(ygat, c_flat)


_BS = 8


def _edge_layer(xs, wa, wb, b):
    o = wa.shape[1]
    go = max(o, 128)
    n = _BS * _P
    if isinstance(xs, list):
        tops = [_edge_topk(xh, wa, wb, b, 0, _BS) for xh in xs]
    else:
        tops = [_edge_topk(xs, wa, wb, b, h, _BS)
                for h in range(0, _B, _BS)]
    gats = [_sc_gather(y.reshape(n, go), idx.reshape(n * _K))
            for idx, y, _ in tops]
    return [
        _nbr_max(yg.reshape(n, _K * go), c.reshape(n, o), go)
        .reshape(_BS, _P, o)
        for yg, (_, _, c) in zip(gats, tops)
    ]


def _fc1_kernel(x1_ref, x2_ref, x3_ref, wf_a_ref, wf_b_ref, wf_c_ref,
                bf_ref, out_ref):
    h = lax.dot_general(x1_ref[0], wf_a_ref[...], (((1,), (0,)), ((), ())),
                        preferred_element_type=jnp.float32)
    h += lax.dot_general(x2_ref[0], wf_b_ref[...], (((1,), (0,)), ((), ())),
                         preferred_element_type=jnp.float32)
    h += lax.dot_general(x3_ref[0], wf_c_ref[...], (((1,), (0,)), ((), ())),
                         preferred_element_type=jnp.float32)
    h += bf_ref[...]
    out_ref[0] = jnp.max(h, axis=0, keepdims=True)


def _head_kernel(g_ref, wa_ref, ba_ref, wb_ref, bb_ref, wc_ref, bc_ref,
                 out_ref):
    o1 = lax.dot_general(g_ref[...], wa_ref[...], (((1,), (0,)), ((), ())),
                         preferred_element_type=jnp.float32) + ba_ref[...]
    o1 = jnp.maximum(o1, 0.0)
    o2 = lax.dot_general(o1, wb_ref[...], (((1,), (0,)), ((), ())),
                         preferred_element_type=jnp.float32) + bb_ref[...]
    o2 = jnp.maximum(o2, 0.0)
    o3 = lax.dot_general(o2, wc_ref[...], (((1,), (0,)), ((), ())),
                         preferred_element_type=jnp.float32) + bc_ref[...]
    m = jnp.max(o3, axis=1, keepdims=True)
    shifted = o3 - m
    lse = jnp.log(jnp.sum(jnp.exp(shifted), axis=1, keepdims=True))
    out_ref[...] = shifted - lse


def kernel(pos, batch, W1, b1, W2, b2, W3, b3, Wf1, bf1, Wf2a, bf2a,
           Wf2b, bf2b, Wf2c, bf2c):
    del batch
    x0 = pos.reshape(_B, _P, 3)
    x1 = _edge_layer(x0, W1[:3], W1[3:], b1.reshape(1, -1))
    x2 = _edge_layer(x1, W2[:64], W2[64:], b2.reshape(1, -1))
    x3 = _edge_layer(x2, W3[:128], W3[128:], b3.reshape(1, -1))

    g_halves = []
    for x1h, x2h, x3h in zip(x1, x2, x3):
        gh = pl.pallas_call(
            _fc1_kernel,
            grid=(_BS,),
            in_specs=[
                pl.BlockSpec((1, _P, 64), lambda i: (i, 0, 0)),
                pl.BlockSpec((1, _P, 128), lambda i: (i, 0, 0)),
                pl.BlockSpec((1, _P, 256), lambda i: (i, 0, 0)),
                pl.BlockSpec((64, 1024), lambda i: (0, 0)),
                pl.BlockSpec((128, 1024), lambda i: (0, 0)),
                pl.BlockSpec((256, 1024), lambda i: (0, 0)),
                pl.BlockSpec((1, 1024), lambda i: (0, 0)),
            ],
            out_specs=pl.BlockSpec((1, 1, 1024), lambda i: (i, 0, 0)),
            out_shape=jax.ShapeDtypeStruct((_BS, 1, 1024), jnp.float32),
        )(x1h, x2h, x3h, Wf1[:64], Wf1[64:192], Wf1[192:],
          bf1.reshape(1, -1))
        g_halves.append(gh.reshape(_BS, 1024))
    g = jnp.concatenate(g_halves, axis=0)

    return pl.pallas_call(
        _head_kernel,
        out_shape=jax.ShapeDtypeStruct((_B, _OUT), jnp.float32),
    )(g, Wf2a, bf2a.reshape(1, -1), Wf2b, bf2b.reshape(1, -1),
      Wf2c, bf2c.reshape(1, -1))


# speedup vs baseline: 1.8692x; 1.8692x over previous
import functools

import jax
import jax.numpy as jnp
from jax import lax
from jax.experimental import pallas as pl

_B, _P, _K, _OUT = 16, 1024, 20, 40
_NEG = -3e38


def _edge_kernel(x_ref, wa_ref, wb_ref, b_ref, out_ref):
    x = x_ref[0]
    n2 = jnp.sum(x * x, axis=1, keepdims=True)
    gram = lax.dot_general(x, x, (((1,), (1,)), ((), ())),
                           preferred_element_type=jnp.float32)
    ones_row = jnp.ones((1, x.shape[1]), jnp.float32)
    n2r = lax.dot_general(ones_row, x * x, (((1,), (1,)), ((), ())),
                          preferred_element_type=jnp.float32)
    dist = n2 + n2r - 2.0 * gram

    wb = wb_ref[...]
    y = lax.dot_general(x, wb, (((1,), (0,)), ((), ())),
                        preferred_element_type=jnp.float32)
    c = lax.dot_general(x, wa_ref[...] - wb, (((1,), (0,)), ((), ())),
                        preferred_element_type=jnp.float32) + b_ref[...]

    col = lax.broadcasted_iota(jnp.int32, (_P, _P), 1)
    v = jnp.minimum(dist, 254.0) + 1.0
    ib = lax.bitcast_convert_type(v, jnp.int32)
    key = (((ib - 0x3F800000) << 4) & ~1023) | col
    acc = jnp.full(y.shape, _NEG, jnp.float32)
    for _ in range(_K):
        kmin = jnp.min(key, axis=1, keepdims=True)
        sel = key == kmin
        onehot = jnp.where(sel, 1.0, 0.0).astype(jnp.float32)
        picked = lax.dot_general(onehot, y, (((1,), (0,)), ((), ())),
                                 preferred_element_type=jnp.float32)
        acc = jnp.maximum(acc, picked)
        key = jnp.where(sel, jnp.int32(0x7FFFFFFF), key)
    out_ref[0] = c + acc


def _edge_layer(x, wa, wb, b):
    d, o = wa.shape
    return pl.pallas_call(
        _edge_kernel,
        grid=(_B,),
        in_specs=[
            pl.BlockSpec((1, _P, d), lambda i: (i, 0, 0)),
            pl.BlockSpec((d, o), lambda i: (0, 0)),
            pl.BlockSpec((d, o), lambda i: (0, 0)),
            pl.BlockSpec((1, o), lambda i: (0, 0)),
        ],
        out_specs=pl.BlockSpec((1, _P, o), lambda i: (i, 0, 0)),
        out_shape=jax.ShapeDtypeStruct((_B, _P, o), jnp.float32),
    )(x, wa, wb, b)


def _fc1_kernel(x1_ref, x2_ref, x3_ref, wf_a_ref, wf_b_ref, wf_c_ref,
                bf_ref, out_ref):
    h = lax.dot_general(x1_ref[0], wf_a_ref[...], (((1,), (0,)), ((), ())),
                        preferred_element_type=jnp.float32)
    h += lax.dot_general(x2_ref[0], wf_b_ref[...], (((1,), (0,)), ((), ())),
                         preferred_element_type=jnp.float32)
    h += lax.dot_general(x3_ref[0], wf_c_ref[...], (((1,), (0,)), ((), ())),
                         preferred_element_type=jnp.float32)
    h += bf_ref[...]
    out_ref[0] = jnp.max(h, axis=0, keepdims=True)


def _head_kernel(g_ref, wa_ref, ba_ref, wb_ref, bb_ref, wc_ref, bc_ref,
                 out_ref):
    o1 = lax.dot_general(g_ref[...], wa_ref[...], (((1,), (0,)), ((), ())),
                         preferred_element_type=jnp.float32) + ba_ref[...]
    o1 = jnp.maximum(o1, 0.0)
    o2 = lax.dot_general(o1, wb_ref[...], (((1,), (0,)), ((), ())),
                         preferred_element_type=jnp.float32) + bb_ref[...]
    o2 = jnp.maximum(o2, 0.0)
    o3 = lax.dot_general(o2, wc_ref[...], (((1,), (0,)), ((), ())),
                         preferred_element_type=jnp.float32) + bc_ref[...]
    m = jnp.max(o3, axis=1, keepdims=True)
    shifted = o3 - m
    lse = jnp.log(jnp.sum(jnp.exp(shifted), axis=1, keepdims=True))
    out_ref[...] = shifted - lse


def kernel(pos, batch, W1, b1, W2, b2, W3, b3, Wf1, bf1, Wf2a, bf2a,
           Wf2b, bf2b, Wf2c, bf2c):
    del batch
    x0 = pos.reshape(_B, _P, 3)
    x1 = _edge_layer(x0, W1[:3], W1[3:], b1.reshape(1, -1))
    x2 = _edge_layer(x1, W2[:64], W2[64:], b2.reshape(1, -1))
    x3 = _edge_layer(x2, W3[:128], W3[128:], b3.reshape(1, -1))

    g = pl.pallas_call(
        _fc1_kernel,
        grid=(_B,),
        in_specs=[
            pl.BlockSpec((1, _P, 64), lambda i: (i, 0, 0)),
            pl.BlockSpec((1, _P, 128), lambda i: (i, 0, 0)),
            pl.BlockSpec((1, _P, 256), lambda i: (i, 0, 0)),
            pl.BlockSpec((64, 1024), lambda i: (0, 0)),
            pl.BlockSpec((128, 1024), lambda i: (0, 0)),
            pl.BlockSpec((256, 1024), lambda i: (0, 0)),
            pl.BlockSpec((1, 1024), lambda i: (0, 0)),
        ],
        out_specs=pl.BlockSpec((1, 1, 1024), lambda i: (i, 0, 0)),
        out_shape=jax.ShapeDtypeStruct((_B, 1, 1024), jnp.float32),
    )(x1, x2, x3, Wf1[:64], Wf1[64:192], Wf1[192:], bf1.reshape(1, -1))
    g = g.reshape(_B, 1024)

    return pl.pallas_call(
        _head_kernel,
        out_shape=jax.ShapeDtypeStruct((_B, _OUT), jnp.float32),
    )(g, Wf2a, bf2a.reshape(1, -1), Wf2b, bf2b.reshape(1, -1),
      Wf2c, bf2c.reshape(1, -1))
